# baseline (speedup 1.0000x reference)
import functools

import jax
import jax.numpy as jnp
from jax import lax
from jax.experimental import pallas as pl
from jax.experimental.pallas import tpu as pltpu
from jax.experimental.pallas import tpu_sc as plsc

BATCH = 16384
HIST = 50
EMBED_DIM = 64
PADDED_DIM = 128

NC = 2
NS = 16
NW = NC * NS
BPW = BATCH // NW
NBUF = 8
GLEAD = 5
SLEAD = NBUF - GLEAD
NT = BPW // NBUF

_mesh = plsc.VectorSubcoreMesh(core_axis_name="c", subcore_axis_name="s")


@functools.partial(
    pl.kernel,
    out_type=jax.ShapeDtypeStruct((BATCH, HIST, PADDED_DIM), jnp.float32),
    mesh=_mesh,
    scratch_types=[
        pltpu.VMEM((BPW, HIST), jnp.int32),
        pltpu.VMEM((NBUF, HIST, EMBED_DIM), jnp.float32),
    ]
    + [pltpu.SemaphoreType.DMA] * (2 * NBUF),
    compiler_params=pltpu.CompilerParams(use_tc_tiling_on_sc=False),
)
def _sc_gather(table_hbm, idx_hbm, out_hbm, idx_v, bufs, *sems):
    gsem = sems[:NBUF]
    ssem = sems[NBUF:]
    wid = lax.axis_index("s") * NC + lax.axis_index("c")
    b0 = wid * BPW

    pltpu.sync_copy(idx_hbm.at[pl.ds(b0, BPW)], idx_v)

    for b in range(GLEAD):
        pltpu.async_copy(table_hbm.at[idx_v.at[b]], bufs.at[b], gsem[b])

    def step(t, carry):
        for b in range(NBUF):
            r = t * NBUF + b
            pltpu.make_async_copy(
                table_hbm.at[idx_v.at[0]], bufs.at[b], gsem[b]
            ).wait()
            pltpu.async_copy(
                bufs.at[b], out_hbm.at[b0 + r, :, pl.ds(0, EMBED_DIM)], ssem[b]
            )
            bn = (b + GLEAD) % NBUF

            @pl.when(r >= SLEAD)
            def _():
                pltpu.make_async_copy(
                    bufs.at[bn], out_hbm.at[0, :, pl.ds(0, EMBED_DIM)], ssem[bn]
                ).wait()

            @pl.when(r + GLEAD < BPW)
            def _():
                pltpu.async_copy(
                    table_hbm.at[idx_v.at[r + GLEAD]], bufs.at[bn], gsem[bn]
                )

        return carry

    lax.fori_loop(0, NT, step, 0)

    for b in range(GLEAD, NBUF):
        pltpu.make_async_copy(
            bufs.at[b], out_hbm.at[0, :, pl.ds(0, EMBED_DIM)], ssem[b]
        ).wait()


def kernel(predicate_ids, table):
    idx = predicate_ids.astype(jnp.int32)
    return _sc_gather(table, idx)[:, :, :EMBED_DIM]


# speedup vs baseline: 1.3287x; 1.3287x over previous
import functools

import jax
import jax.numpy as jnp
from jax import lax
from jax.experimental import pallas as pl
from jax.experimental.pallas import tpu as pltpu
from jax.experimental.pallas import tpu_sc as plsc

BATCH = 16384
NUM_ROWS = 1000000
HIST = 50
EMBED_DIM = 64
PADDED_DIM = 128

NC = 2
NS = 16
NW = NC * NS
BPW = BATCH // NW
NBUF = 8
GLEAD = 5
SLEAD = NBUF - GLEAD
NT = BPW // NBUF

_mesh = plsc.VectorSubcoreMesh(core_axis_name="c", subcore_axis_name="s")


@functools.partial(
    pl.kernel,
    out_type=jax.ShapeDtypeStruct((BATCH, HIST, PADDED_DIM), jnp.float32),
    mesh=_mesh,
    scratch_types=[
        pltpu.VMEM((BPW, HIST), jnp.int32),
        pltpu.VMEM((NBUF, HIST, PADDED_DIM), jnp.float32),
    ]
    + [pltpu.SemaphoreType.DMA] * (2 * NBUF),
    compiler_params=pltpu.CompilerParams(use_tc_tiling_on_sc=True),
)
def _sc_gather(table_hbm, idx_hbm, out_hbm, idx_v, bufs, *sems):
    gsem = sems[:NBUF]
    ssem = sems[NBUF:]
    wid = lax.axis_index("s") * NC + lax.axis_index("c")
    b0 = wid * BPW

    pltpu.sync_copy(idx_hbm.at[pl.ds(b0, BPW)], idx_v)

    for b in range(GLEAD):
        pltpu.async_copy(table_hbm.at[idx_v.at[b]], bufs.at[b], gsem[b])

    def step(t, carry):
        for b in range(NBUF):
            r = t * NBUF + b
            pltpu.make_async_copy(
                table_hbm.at[idx_v.at[0]], bufs.at[b], gsem[b]
            ).wait()
            pltpu.async_copy(bufs.at[b], out_hbm.at[b0 + r], ssem[b])
            bn = (b + GLEAD) % NBUF

            @pl.when(r >= SLEAD)
            def _():
                pltpu.make_async_copy(
                    bufs.at[bn], out_hbm.at[0], ssem[bn]
                ).wait()

            @pl.when(r + GLEAD < BPW)
            def _():
                pltpu.async_copy(
                    table_hbm.at[idx_v.at[r + GLEAD]], bufs.at[bn], gsem[bn]
                )

        return carry

    lax.fori_loop(0, NT, step, 0)

    for b in range(GLEAD, NBUF):
        pltpu.make_async_copy(bufs.at[b], out_hbm.at[0], ssem[b]).wait()


def kernel(predicate_ids, table):
    idx = predicate_ids.astype(jnp.int32)
    tpad = jax.lax.dynamic_update_slice(
        jnp.zeros((NUM_ROWS, PADDED_DIM), jnp.float32), table, (0, 0)
    )
    return _sc_gather(tpad, idx)[:, :, :EMBED_DIM]


# speedup vs baseline: 1.3314x; 1.0020x over previous
import functools

import jax
import jax.numpy as jnp
from jax import lax
from jax.experimental import pallas as pl
from jax.experimental.pallas import tpu as pltpu
from jax.experimental.pallas import tpu_sc as plsc

BATCH = 16384
NUM_ROWS = 1000000
HIST = 50
EMBED_DIM = 64
PADDED_DIM = 128

NC = 2
NS = 16
NW = NC * NS
BPW = BATCH // NW
NBUF = 8
GLEAD = 6
SLEAD = NBUF - GLEAD
NT = BPW // NBUF

_mesh = plsc.VectorSubcoreMesh(core_axis_name="c", subcore_axis_name="s")


@functools.partial(
    pl.kernel,
    out_type=jax.ShapeDtypeStruct((BATCH, HIST, PADDED_DIM), jnp.float32),
    mesh=_mesh,
    scratch_types=[
        pltpu.VMEM((BPW, HIST), jnp.int32),
        pltpu.VMEM((NBUF, HIST, PADDED_DIM), jnp.float32),
    ]
    + [pltpu.SemaphoreType.DMA] * (2 * NBUF),
    compiler_params=pltpu.CompilerParams(use_tc_tiling_on_sc=True),
)
def _sc_gather(table_hbm, idx_hbm, out_hbm, idx_v, bufs, *sems):
    gsem = sems[:NBUF]
    ssem = sems[NBUF:]
    wid = lax.axis_index("s") * NC + lax.axis_index("c")
    b0 = wid * BPW

    pltpu.sync_copy(idx_hbm.at[pl.ds(b0, BPW)], idx_v)

    for b in range(GLEAD):
        pltpu.async_copy(table_hbm.at[idx_v.at[b]], bufs.at[b], gsem[b])

    def step(t, carry):
        for b in range(NBUF):
            r = t * NBUF + b
            pltpu.make_async_copy(
                table_hbm.at[idx_v.at[0]], bufs.at[b], gsem[b]
            ).wait()
            pltpu.async_copy(bufs.at[b], out_hbm.at[b0 + r], ssem[b])
            bn = (b + GLEAD) % NBUF

            @pl.when(r >= SLEAD)
            def _():
                pltpu.make_async_copy(
                    bufs.at[bn], out_hbm.at[0], ssem[bn]
                ).wait()

            @pl.when(r + GLEAD < BPW)
            def _():
                pltpu.async_copy(
                    table_hbm.at[idx_v.at[r + GLEAD]], bufs.at[bn], gsem[bn]
                )

        return carry

    lax.fori_loop(0, NT, step, 0)

    for b in range(GLEAD, NBUF):
        pltpu.make_async_copy(bufs.at[b], out_hbm.at[0], ssem[b]).wait()


def kernel(predicate_ids, table):
    idx = predicate_ids.astype(jnp.int32)
    tpad = jax.lax.dynamic_update_slice(
        jnp.zeros((NUM_ROWS, PADDED_DIM), jnp.float32), table, (0, 0)
    )
    return _sc_gather(tpad, idx)[:, :, :EMBED_DIM]
